# Initial kernel scaffold; baseline (speedup 1.0000x reference)
import jax
import jax.numpy as jnp
from jax.experimental import pallas as pl


def kernel(x, E1, E2, E3, W1, b1, W2, b2):
    raise NotImplementedError("write your pallas kernel here")


# speedup vs baseline: 1.3041x; 1.3041x over previous
import functools

import jax
import jax.numpy as jnp
from jax import lax
from jax.experimental import pallas as pl
from jax.experimental.pallas import tpu as pltpu
from jax.experimental.pallas import tpu_sc as plsc

_D = 64
_B = 16384
_DIM = 128

_NC = 2
_NS = 16
_NW = _NC * _NS
_BPW = _B // _NW
_CHUNK = 128
_NCHUNK = _BPW // _CHUNK


def _gather_body(x_hbm, e1_hbm, e2_hbm, e3_hbm, o1, o2, o3, idx_v, rows_v, sem):
    wid = lax.axis_index("s") * _NC + lax.axis_index("c")
    base = wid * _BPW
    cbase = wid * _NCHUNK
    tables = (e1_hbm, e2_hbm, e3_hbm)
    outs = (o1, o2, o3)
    for t in range(3):
        pltpu.sync_copy(x_hbm.at[t, pl.ds(cbase, _NCHUNK)], idx_v.at[t])
    copies = []
    for t in range(3):
        for j in range(_NCHUNK):
            copies.append(
                pltpu.async_copy(
                    tables[t].at[idx_v.at[t, j]],
                    rows_v.at[t, pl.ds(j * _CHUNK, _CHUNK)],
                    sem,
                )
            )
    for c in copies:
        c.wait()
    for t in range(3):
        pltpu.sync_copy(rows_v.at[t], outs[t].at[pl.ds(base, _BPW)])


def _make_gather(num_labels):
    mesh = plsc.VectorSubcoreMesh(core_axis_name="c", subcore_axis_name="s")
    out = jax.ShapeDtypeStruct((_B, _D), jnp.float32)
    return pl.kernel(
        _gather_body,
        mesh=mesh,
        out_type=(out, out, out),
        scratch_types=[
            pltpu.VMEM((3, _NCHUNK, _CHUNK), jnp.int32),
            pltpu.VMEM((3, _BPW, _D), jnp.float32),
            pltpu.SemaphoreType.DMA,
        ],
        compiler_params=pltpu.CompilerParams(use_tc_tiling_on_sc=False),
    )


def _mlp_body(e1, e2, e3, w1, b1, w2, b2, o):
    h = jnp.dot(e1[...], w1[0:_D, :], preferred_element_type=jnp.float32)
    h = h + jnp.dot(e2[...], w1[_D:2 * _D, :], preferred_element_type=jnp.float32)
    h = h + jnp.dot(e3[...], w1[2 * _D:3 * _D, :], preferred_element_type=jnp.float32)
    h = h + b1[...]
    h = h * jax.nn.sigmoid(h)
    o[...] = jnp.dot(h, w2[...], preferred_element_type=jnp.float32) + b2[...]


def _mlp_call(e1, e2, e3, W1, b1, W2, b2):
    r = 2048
    espec = pl.BlockSpec((r, _D), lambda i: (i, 0))
    full = lambda s: pl.BlockSpec(s, lambda i: (0, 0))
    return pl.pallas_call(
        _mlp_body,
        grid=(_B // r,),
        in_specs=[espec, espec, espec,
                  full((3 * _D, _DIM)), full((1, _DIM)),
                  full((_DIM, _DIM)), full((1, _DIM))],
        out_specs=pl.BlockSpec((r, _DIM), lambda i: (i, 0)),
        out_shape=jax.ShapeDtypeStruct((_B, _DIM), jnp.float32),
    )(e1, e2, e3, W1, b1.reshape(1, _DIM), W2, b2.reshape(1, _DIM))


def kernel(x, E1, E2, E3, W1, b1, W2, b2):
    x_r = x.astype(jnp.int32).reshape(3, _B // _CHUNK, _CHUNK)
    gather = _make_gather(E1.shape[0])
    e1, e2, e3 = gather(x_r, E1, E2, E3)
    return _mlp_call(e1, e2, e3, W1, b1, W2, b2)
